# baseline (speedup 1.0000x reference)
import numpy as np
import jax
import jax.numpy as jnp
from jax import lax
from jax.experimental import pallas as pl
from jax.experimental.pallas import tpu as pltpu
from jax.experimental.pallas import tpu_sc as plsc

_KEEP_RATIO = 0.7
_N = 65536
_NT = 16
_ROWS = _N // _NT
_CHUNK = 512
_NCHUNK = _ROWS // _CHUNK
_L = 16
_B = 256
_CB = _ROWS + _L

_KTAB_PAD = _N + 16
_KTAB_NP = np.zeros((_KTAB_PAD,), np.int32)
_KTAB_NP[: _N + 1] = (
    np.arange(_N + 1, dtype=np.float64) * _KEEP_RATIO
).astype(np.int32)

_F32 = jnp.float32
_I32 = jnp.int32


def _sc_body(clss_h, cp_h, bt_h, bp_h, lt_h, lp_h, ktab_h, out_h,
             clss_b, cp_b, bt_b, bp_b, lt_b, lp_b,
             clss_b2, cp_b2, bt_b2, bp_b2, lt_b2, lp_b2,
             kall, cb, hist, hist3, comb, stg_i, stg_f,
             tileh, cnt_rb, sbuf, dec3_b, ktb,
             hists_s, counts_s, sums_s, decs_s,
             sem, sem2):
    wid = lax.axis_index("s")
    iota = lax.iota(_I32, _L)
    zero_i = jnp.zeros((_L,), _I32)
    ones_i = jnp.full((_L,), 1, _I32)
    fzero = jnp.zeros((_L,), _F32)

    def lane(vec, j):
        return jnp.sum(jnp.where(iota == j, vec, jnp.zeros_like(vec)))

    def set_lane(vec, j, val):
        return jnp.where(iota == j, val, vec)

    set0 = (clss_b, cp_b, bt_b, bp_b, lt_b, lp_b)
    set1 = (clss_b2, cp_b2, bt_b2, bp_b2, lt_b2, lp_b2)

    def issue(c, bufs, sm):
        base = wid * _ROWS + c * _CHUNK
        pltpu.async_copy(clss_h.at[pl.ds(base, _CHUNK)], bufs[0], sm)
        pltpu.async_copy(cp_h.at[:, pl.ds(base, _CHUNK)], bufs[1], sm)
        pltpu.async_copy(bt_h.at[:, pl.ds(base, _CHUNK)], bufs[2], sm)
        pltpu.async_copy(bp_h.at[:, pl.ds(base, _CHUNK)], bufs[3], sm)
        pltpu.async_copy(lt_h.at[:, pl.ds(base, _CHUNK)], bufs[4], sm)
        pltpu.async_copy(lp_h.at[:, pl.ds(base, _CHUNK)], bufs[5], sm)

    def drain(c, bufs, sm):
        base = wid * _ROWS + c * _CHUNK
        pltpu.make_async_copy(clss_h.at[pl.ds(base, _CHUNK)], bufs[0], sm).wait()
        pltpu.make_async_copy(cp_h.at[:, pl.ds(base, _CHUNK)], bufs[1], sm).wait()
        pltpu.make_async_copy(bt_h.at[:, pl.ds(base, _CHUNK)], bufs[2], sm).wait()
        pltpu.make_async_copy(bp_h.at[:, pl.ds(base, _CHUNK)], bufs[3], sm).wait()
        pltpu.make_async_copy(lt_h.at[:, pl.ds(base, _CHUNK)], bufs[4], sm).wait()
        pltpu.make_async_copy(lp_h.at[:, pl.ds(base, _CHUNK)], bufs[5], sm).wait()

    def compute_chunk(c, bufs, carry):
        clss_x, cp_x, bt_x, bp_x, lt_x, lp_x = bufs

        def vec_body(i, carry2):
            cnt_c, cnt_b, cnt_l = carry2
            sl = pl.ds(i * _L, _L)
            cv = clss_x[sl]
            p0 = cp_x[0, sl]
            p1 = cp_x[1, sl]
            m = jnp.maximum(p0, p1)
            z = jnp.exp(p0 - m) + jnp.exp(p1 - m)
            s = (z - _F32(1.0)) / (z + _F32(1.0))
            u = s * s
            poly = jnp.full((_L,), 1.0 / 15.0, _F32)
            for coef in (1.0 / 13.0, 1.0 / 11.0, 1.0 / 9.0,
                         1.0 / 7.0, 1.0 / 5.0, 1.0 / 3.0):
                poly = poly * u + _F32(coef)
            logz = _F32(2.0) * s * (poly * u + _F32(1.0))
            psel = jnp.where(cv == 0, p0, p1)
            lc = logz + m - psel
            vc = cv != -1
            key_c = jnp.where(vc, plsc.bitcast(lc, _I32), jnp.full((_L,), -1, _I32))
            accb = fzero
            for j in range(4):
                d = bt_x[j, sl] - bp_x[j, sl]
                accb = accb + d * d
            lb = accb / _F32(4.0)
            vb = cv != 0
            key_b = jnp.where(vb, plsc.bitcast(lb, _I32), jnp.full((_L,), -1, _I32))
            accl = fzero
            for j in range(10):
                d = lt_x[j, sl] - lp_x[j, sl]
                accl = accl + d * d
            ll = accl / _F32(10.0)
            vl = cv == -2
            key_l = jnp.where(vl, plsc.bitcast(ll, _I32), jnp.full((_L,), -1, _I32))

            pos = c * _CHUNK + i * _L
            kall[pl.ds(pos, _L)] = key_c
            kall[pl.ds(_ROWS + pos, _L)] = key_b
            kall[pl.ds(2 * _ROWS + pos, _L)] = key_l
            plsc.addupdate_scatter(
                hist3, [iota * _B + jnp.right_shift(key_c, 23)], ones_i, mask=vc)
            plsc.addupdate_scatter(
                hist3, [(_L * _B) + iota * _B + jnp.right_shift(key_b, 23)],
                ones_i, mask=vb)
            plsc.addupdate_scatter(
                hist3, [(2 * _L * _B) + iota * _B + jnp.right_shift(key_l, 23)],
                ones_i, mask=vl)
            cnt_c = cnt_c + jnp.where(vc, ones_i, zero_i)
            cnt_b = cnt_b + jnp.where(vb, ones_i, zero_i)
            cnt_l = cnt_l + jnp.where(vl, ones_i, zero_i)
            return (cnt_c, cnt_b, cnt_l)

        return lax.fori_loop(0, _CHUNK // _L, vec_body, carry)

    with jax.named_scope("ph0_loss"):
        issue(0, set0, sem)

        def zh3(i, _):
            hist3[pl.ds(i * _L, _L)] = zero_i
            return 0
        lax.fori_loop(0, (3 * _L * _B) // _L, zh3, 0)

        def pair_body(h, carry):
            issue(2 * h + 1, set1, sem2)
            drain(2 * h, set0, sem)
            carry = compute_chunk(2 * h, set0, carry)

            @pl.when(h < _NCHUNK // 2 - 1)
            def _():
                issue(2 * h + 2, set0, sem)

            drain(2 * h + 1, set1, sem2)
            return compute_chunk(2 * h + 1, set1, carry)

        cnt_c, cnt_b, cnt_l = lax.fori_loop(
            0, _NCHUNK // 2, pair_body, (zero_i, zero_i, zero_i))

    cvec = set_lane(set_lane(set_lane(
        zero_i, 0, jnp.sum(cnt_c)), 1, jnp.sum(cnt_b)), 2, jnp.sum(cnt_l))
    stg_i[...] = cvec
    pltpu.sync_copy(stg_i, counts_s.at[pl.ds(wid * _L, _L)])

    def zero_hist():
        def zb(i, _):
            hist[pl.ds(i * _L, _L)] = zero_i
            return 0
        lax.fori_loop(0, (_L * _B) // _L, zb, 0)

    def combine_16x256(src, dst):
        def cbdy(v, _):
            acc = src[pl.ds(v * _L, _L)]
            for t in range(1, _NT):
                acc = acc + src[pl.ds(t * _B + v * _L, _L)]
            dst[pl.ds(v * _L, _L)] = acc
            return 0
        lax.fori_loop(0, _B // _L, cbdy, 0)

    def publish(j):
        pltpu.sync_copy(comb, hists_s.at[pl.ds((j * _NT + wid) * _B, _B)])

    def decide(r):
        def scan_body(i, carry):
            c_above, b_star, above_star = carry
            v = (_B // _L - 1) - i
            vec = comb[pl.ds(v * _L, _L)]
            rv = lax.rev(vec, (0,))
            inc = plsc.cumsum(rv)
            above = c_above + inc - rv
            hit = jnp.logical_and(above < r, above + rv >= r)
            ids_desc = v * _L + (_L - 1) - iota
            b_star = b_star + jnp.sum(jnp.where(hit, ids_desc, zero_i))
            above_star = above_star + jnp.sum(jnp.where(hit, above, zero_i))
            c_above = c_above + jnp.sum(vec)
            return (c_above, b_star, above_star)

        _, b_star, above_star = lax.fori_loop(
            0, _B // _L, scan_body, (_I32(0), _I32(0), _I32(0)))
        return b_star, above_star

    def fetch_and_combine(j):
        pltpu.sync_copy(hists_s.at[pl.ds(j * _NT * _B, _NT * _B)], tileh)
        combine_16x256(tileh, comb)

    def write_dec(j, prefix, r, k):
        stg_i[...] = set_lane(set_lane(set_lane(
            zero_i, 0, prefix), 1, r), 2, k)
        pltpu.sync_copy(stg_i, decs_s.at[pl.ds(j * _L, _L)])

    def read_decs():
        pltpu.sync_copy(decs_s, dec3_b)
        rows = [dec3_b[pl.ds(j * _L, _L)] for j in range(3)]
        return ([lane(rows[j], 0) for j in range(3)],
                [lane(rows[j], 1) for j in range(3)],
                [lane(rows[j], 2) for j in range(3)])

    def lookup_k(nv):
        base = pl.multiple_of(jnp.bitwise_and(nv, _I32(-8)), 8)
        pltpu.sync_copy(ktab_h.at[pl.ds(base, _L)], ktb)
        return lane(ktb[...], nv - base)

    def p1_body(j, _):
        def cbdy(v, _2):
            acc = hist3[pl.ds(j * (_L * _B) + v * _L, _L)]
            for t in range(1, _NT):
                acc = acc + hist3[pl.ds(j * (_L * _B) + t * _B + v * _L, _L)]
            comb[pl.ds(v * _L, _L)] = acc
            return 0
        lax.fori_loop(0, _B // _L, cbdy, 0)
        publish(j)
        return 0
    with jax.named_scope("ph1_hist"):
        lax.fori_loop(0, 3, p1_body, 0)
    plsc.subcore_barrier()

    @pl.when(wid < 3)
    def _():
        j = wid
        pltpu.sync_copy(counts_s, cnt_rb)
        cacc = cnt_rb[pl.ds(0, _L)]
        for t in range(1, _NT):
            cacc = cacc + cnt_rb[pl.ds(t * _L, _L)]
        nv = lane(cacc, j)
        k = lookup_k(nv)
        fetch_and_combine(j)
        b_star, above_star = decide(k)
        write_dec(j, b_star, k - above_star, k)

    zero_hist()
    plsc.subcore_barrier()
    p1s, r1s, k1s = read_decs()

    wcnts = []
    sumhis = []
    _ns2 = jax.named_scope("ph2_hist"); _ns2.__enter__()
    for j in range(3):
        p1j = p1s[j]

        def sb2(i, carry):
            wcnt, shi = carry
            kv = kall[pl.ds(j * _ROWS + i * _L, _L)]
            d1 = jnp.right_shift(kv, 23)
            msk = d1 == p1j
            plsc.store_compressed(cb.at[pl.ds(j * _CB + wcnt, _L)], kv, mask=msk)
            wcnt = wcnt + plsc.all_reduce_population_count(msk)[0]
            shi = shi + jnp.where(d1 > p1j, plsc.bitcast(kv, _F32), fzero)
            return (wcnt, shi)

        wcnt, shi = lax.fori_loop(0, _ROWS // _L, sb2, (_I32(0), fzero))
        cb[pl.ds(j * _CB + wcnt, _L)] = jnp.full((_L,), -1, _I32)
        wcnts.append(wcnt)
        sumhis.append(shi)
    _ns2.__exit__(None, None, None)

    def hist_compacted(j, pj, shift, width):
        trip = jnp.right_shift(wcnts[j] + (_L - 1), 4)
        pshift = shift + width

        def sbc(i, _2):
            kv = cb[pl.ds(j * _CB + i * _L, _L)]
            msk = jnp.right_shift(kv, pshift) == pj
            digit = jnp.bitwise_and(jnp.right_shift(kv, shift), (1 << width) - 1)
            plsc.addupdate_scatter(hist, [iota * _B + digit], ones_i, mask=msk)
            return 0

        lax.fori_loop(0, trip, sbc, 0)
        combine_16x256(hist, comb)
        publish(j)

    for j in range(3):
        hist_compacted(j, p1s[j], 15, 8)
        if j < 2:
            zero_hist()
    plsc.subcore_barrier()

    def decide_round(width):
        @pl.when(wid < 3)
        def _():
            j = wid
            row = dec3_b[pl.ds(wid * _L, _L)]
            p = lane(row, 0)
            r = lane(row, 1)
            k = lane(row, 2)
            fetch_and_combine(j)
            b_star, above_star = decide(r)
            write_dec(j, p * _I32(1 << width) + b_star, r - above_star, k)

    decide_round(8)
    zero_hist()
    plsc.subcore_barrier()
    p2s, r2s, k2s = read_decs()

    for (sh, w) in ((7, 8), (0, 7)):
        for j in range(3):
            hist_compacted(j, p2s[j], sh, w)
            if j < 2:
                zero_hist()
        plsc.subcore_barrier()
        decide_round(w)
        if sh == 7:
            zero_hist()
        plsc.subcore_barrier()
        p2s, r2s, k2s = read_decs()

    ts, rfs, kfs = p2s, r2s, k2s

    svec = fzero
    for j in range(3):
        tj = ts[j]
        trip = jnp.right_shift(wcnts[j] + (_L - 1), 4)

        def sfin(i, acc, j=j, tj=tj):
            kv = cb[pl.ds(j * _CB + i * _L, _L)]
            return acc + jnp.where(kv > tj, plsc.bitcast(kv, _F32), fzero)

        acc = lax.fori_loop(0, trip, sfin, sumhis[j])
        svec = set_lane(svec, j, jnp.sum(acc))
    stg_f[...] = svec
    pltpu.sync_copy(stg_f, sums_s.at[pl.ds(wid * _L, _L)])
    plsc.subcore_barrier()

    @pl.when(wid == 0)
    def _(ts=ts, rfs=rfs, kfs=kfs):
        pltpu.sync_copy(sums_s, sbuf)
        sacc = sbuf[pl.ds(0, _L)]
        for t in range(1, _NT):
            sacc = sacc + sbuf[pl.ds(t * _L, _L)]
        tvec = zero_i
        rvec = zero_i
        kvec = zero_i
        for j in range(3):
            tvec = set_lane(tvec, j, ts[j])
            rvec = set_lane(rvec, j, rfs[j])
            kvec = set_lane(kvec, j, kfs[j])
        total = sacc + rvec.astype(_F32) * plsc.bitcast(tvec, _F32)
        res = total / kvec.astype(_F32)
        res = jnp.where(kvec > 0, res, jnp.full((_L,), jnp.nan, _F32))
        stg_f[...] = res
        pltpu.sync_copy(stg_f, out_h)


def _make_call():
    mesh = plsc.VectorSubcoreMesh(
        core_axis_name="c", subcore_axis_name="s", num_cores=1)
    return pl.kernel(
        _sc_body,
        out_type=jax.ShapeDtypeStruct((_L,), jnp.float32),
        mesh=mesh,
        scratch_types=[
            pltpu.VMEM((_CHUNK,), _I32),
            pltpu.VMEM((2, _CHUNK), _F32),
            pltpu.VMEM((4, _CHUNK), _F32),
            pltpu.VMEM((4, _CHUNK), _F32),
            pltpu.VMEM((10, _CHUNK), _F32),
            pltpu.VMEM((10, _CHUNK), _F32),
            pltpu.VMEM((_CHUNK,), _I32),
            pltpu.VMEM((2, _CHUNK), _F32),
            pltpu.VMEM((4, _CHUNK), _F32),
            pltpu.VMEM((4, _CHUNK), _F32),
            pltpu.VMEM((10, _CHUNK), _F32),
            pltpu.VMEM((10, _CHUNK), _F32),
            pltpu.VMEM((3 * _ROWS,), _I32),
            pltpu.VMEM((3 * _CB,), _I32),
            pltpu.VMEM((_L * _B,), _I32),
            pltpu.VMEM((3 * _L * _B,), _I32),
            pltpu.VMEM((_B,), _I32),
            pltpu.VMEM((_L,), _I32),
            pltpu.VMEM((_L,), _F32),
            pltpu.VMEM((_NT * _B,), _I32),
            pltpu.VMEM((_NT * _L,), _I32),
            pltpu.VMEM((_NT * _L,), _F32),
            pltpu.VMEM((3 * _L,), _I32),
            pltpu.VMEM((_L,), _I32),
            pltpu.VMEM_SHARED((3 * _NT * _B,), _I32),
            pltpu.VMEM_SHARED((_NT * _L,), _I32),
            pltpu.VMEM_SHARED((_NT * _L,), _F32),
            pltpu.VMEM_SHARED((3 * _L,), _I32),
            pltpu.SemaphoreType.DMA,
            pltpu.SemaphoreType.DMA,
        ],
        compiler_params=pltpu.CompilerParams(
            needs_layout_passes=False, use_tc_tiling_on_sc=True),
    )


_sc_call = _make_call()


@jax.jit
def kernel(clss, cls_pred, bbox_true, bbox_pred, ldmk_true, ldmk_pred):
    ktab = jnp.asarray(_KTAB_NP)
    out = _sc_call(
        clss.astype(_I32),
        cls_pred.astype(_F32).T,
        bbox_true.astype(_F32).T,
        bbox_pred.astype(_F32).T,
        ldmk_true.astype(_F32).T,
        ldmk_pred.astype(_F32).T,
        ktab,
    )
    return out[:3]


# speedup vs baseline: 1.0301x; 1.0301x over previous
import numpy as np
import jax
import jax.numpy as jnp
from jax import lax
from jax.experimental import pallas as pl
from jax.experimental.pallas import tpu as pltpu
from jax.experimental.pallas import tpu_sc as plsc

_KEEP_RATIO = 0.7
_N = 65536
_NT = 16
_ROWS = _N // _NT
_CHUNK = 512
_NCHUNK = _ROWS // _CHUNK
_L = 16
_B = 256
_CB = _ROWS + _L

_KTAB_PAD = _N + 16
_KTAB_NP = np.zeros((_KTAB_PAD,), np.int32)
_KTAB_NP[: _N + 1] = (
    np.arange(_N + 1, dtype=np.float64) * _KEEP_RATIO
).astype(np.int32)

_F32 = jnp.float32
_I32 = jnp.int32


def _sc_body(clss_h, cp_h, bt_h, bp_h, lt_h, lp_h, ktab_h, out_h,
             clss_b, cp_b, bt_b, bp_b, lt_b, lp_b,
             clss_b2, cp_b2, bt_b2, bp_b2, lt_b2, lp_b2,
             kall, cb, hist, hist3, comb, stg_i, stg_f,
             tileh, cnt_rb, sbuf, dec3_b, ktb,
             hists_s, counts_s, sums_s, decs_s,
             sem, sem2):
    wid = lax.axis_index("s")
    iota = lax.iota(_I32, _L)
    zero_i = jnp.zeros((_L,), _I32)
    ones_i = jnp.full((_L,), 1, _I32)
    fzero = jnp.zeros((_L,), _F32)

    def lane(vec, j):
        return jnp.sum(jnp.where(iota == j, vec, jnp.zeros_like(vec)))

    def set_lane(vec, j, val):
        return jnp.where(iota == j, val, vec)

    set0 = (clss_b, cp_b, bt_b, bp_b, lt_b, lp_b)
    set1 = (clss_b2, cp_b2, bt_b2, bp_b2, lt_b2, lp_b2)

    def issue(c, bufs, sm):
        base = wid * _ROWS + c * _CHUNK
        pltpu.async_copy(clss_h.at[pl.ds(base, _CHUNK)], bufs[0], sm)
        pltpu.async_copy(cp_h.at[:, pl.ds(base, _CHUNK)], bufs[1], sm)
        pltpu.async_copy(bt_h.at[:, pl.ds(base, _CHUNK)], bufs[2], sm)
        pltpu.async_copy(bp_h.at[:, pl.ds(base, _CHUNK)], bufs[3], sm)
        pltpu.async_copy(lt_h.at[:, pl.ds(base, _CHUNK)], bufs[4], sm)
        pltpu.async_copy(lp_h.at[:, pl.ds(base, _CHUNK)], bufs[5], sm)

    def drain(c, bufs, sm):
        base = wid * _ROWS + c * _CHUNK
        pltpu.make_async_copy(clss_h.at[pl.ds(base, _CHUNK)], bufs[0], sm).wait()
        pltpu.make_async_copy(cp_h.at[:, pl.ds(base, _CHUNK)], bufs[1], sm).wait()
        pltpu.make_async_copy(bt_h.at[:, pl.ds(base, _CHUNK)], bufs[2], sm).wait()
        pltpu.make_async_copy(bp_h.at[:, pl.ds(base, _CHUNK)], bufs[3], sm).wait()
        pltpu.make_async_copy(lt_h.at[:, pl.ds(base, _CHUNK)], bufs[4], sm).wait()
        pltpu.make_async_copy(lp_h.at[:, pl.ds(base, _CHUNK)], bufs[5], sm).wait()

    def compute_chunk(c, bufs, carry):
        clss_x, cp_x, bt_x, bp_x, lt_x, lp_x = bufs

        def vec_body(i, carry2):
            cnt_c, cnt_b, cnt_l = carry2
            sl = pl.ds(i * _L, _L)
            cv = clss_x[sl]
            p0 = cp_x[0, sl]
            p1 = cp_x[1, sl]
            m = jnp.maximum(p0, p1)
            z = jnp.exp(p0 - m) + jnp.exp(p1 - m)
            s = (z - _F32(1.0)) / (z + _F32(1.0))
            u = s * s
            poly = jnp.full((_L,), 1.0 / 15.0, _F32)
            for coef in (1.0 / 13.0, 1.0 / 11.0, 1.0 / 9.0,
                         1.0 / 7.0, 1.0 / 5.0, 1.0 / 3.0):
                poly = poly * u + _F32(coef)
            logz = _F32(2.0) * s * (poly * u + _F32(1.0))
            psel = jnp.where(cv == 0, p0, p1)
            lc = logz + m - psel
            vc = cv != -1
            key_c = jnp.where(vc, plsc.bitcast(lc, _I32), jnp.full((_L,), -1, _I32))
            accb = fzero
            for j in range(4):
                d = bt_x[j, sl] - bp_x[j, sl]
                accb = accb + d * d
            lb = accb / _F32(4.0)
            vb = cv != 0
            key_b = jnp.where(vb, plsc.bitcast(lb, _I32), jnp.full((_L,), -1, _I32))
            accl = fzero
            for j in range(10):
                d = lt_x[j, sl] - lp_x[j, sl]
                accl = accl + d * d
            ll = accl / _F32(10.0)
            vl = cv == -2
            key_l = jnp.where(vl, plsc.bitcast(ll, _I32), jnp.full((_L,), -1, _I32))

            pos = c * _CHUNK + i * _L
            kall[pl.ds(pos, _L)] = key_c
            kall[pl.ds(_ROWS + pos, _L)] = key_b
            kall[pl.ds(2 * _ROWS + pos, _L)] = key_l
            plsc.addupdate_scatter(
                hist3, [iota * _B + jnp.right_shift(key_c, 23)], ones_i, mask=vc)
            plsc.addupdate_scatter(
                hist3, [(_L * _B) + iota * _B + jnp.right_shift(key_b, 23)],
                ones_i, mask=vb)
            plsc.addupdate_scatter(
                hist3, [(2 * _L * _B) + iota * _B + jnp.right_shift(key_l, 23)],
                ones_i, mask=vl)
            cnt_c = cnt_c + jnp.where(vc, ones_i, zero_i)
            cnt_b = cnt_b + jnp.where(vb, ones_i, zero_i)
            cnt_l = cnt_l + jnp.where(vl, ones_i, zero_i)
            return (cnt_c, cnt_b, cnt_l)

        return lax.fori_loop(0, _CHUNK // _L, vec_body, carry)

    with jax.named_scope("ph0_loss"):
        issue(0, set0, sem)

        def zh3(i, _):
            hist3[pl.ds(i * _L, _L)] = zero_i
            return 0
        lax.fori_loop(0, (3 * _L * _B) // _L, zh3, 0)

        def pair_body(h, carry):
            issue(2 * h + 1, set1, sem2)
            drain(2 * h, set0, sem)
            carry = compute_chunk(2 * h, set0, carry)

            @pl.when(h < _NCHUNK // 2 - 1)
            def _():
                issue(2 * h + 2, set0, sem)

            drain(2 * h + 1, set1, sem2)
            return compute_chunk(2 * h + 1, set1, carry)

        cnt_c, cnt_b, cnt_l = lax.fori_loop(
            0, _NCHUNK // 2, pair_body, (zero_i, zero_i, zero_i))

    cvec = set_lane(set_lane(set_lane(
        zero_i, 0, jnp.sum(cnt_c)), 1, jnp.sum(cnt_b)), 2, jnp.sum(cnt_l))
    stg_i[...] = cvec
    pltpu.sync_copy(stg_i, counts_s.at[pl.ds(wid * _L, _L)])

    def zero_hist():
        def zb(i, _):
            hist[pl.ds(i * _L, _L)] = zero_i
            return 0
        lax.fori_loop(0, (_L * _B) // _L, zb, 0)

    def combine_16x256(src, dst):
        def cbdy(v, _):
            acc = src[pl.ds(v * _L, _L)]
            for t in range(1, _NT):
                acc = acc + src[pl.ds(t * _B + v * _L, _L)]
            dst[pl.ds(v * _L, _L)] = acc
            return 0
        lax.fori_loop(0, _B // _L, cbdy, 0)

    def publish(j):
        pltpu.sync_copy(comb, hists_s.at[pl.ds((j * _NT + wid) * _B, _B)])

    def decide(r):
        def scan_body(i, carry):
            c_above, b_star, above_star = carry
            v = (_B // _L - 1) - i
            vec = comb[pl.ds(v * _L, _L)]
            rv = lax.rev(vec, (0,))
            inc = plsc.cumsum(rv)
            above = c_above + inc - rv
            hit = jnp.logical_and(above < r, above + rv >= r)
            ids_desc = v * _L + (_L - 1) - iota
            b_star = b_star + jnp.sum(jnp.where(hit, ids_desc, zero_i))
            above_star = above_star + jnp.sum(jnp.where(hit, above, zero_i))
            c_above = c_above + jnp.sum(vec)
            return (c_above, b_star, above_star)

        _, b_star, above_star = lax.fori_loop(
            0, _B // _L, scan_body, (_I32(0), _I32(0), _I32(0)))
        return b_star, above_star

    def fetch_and_combine(j):
        pltpu.sync_copy(hists_s.at[pl.ds(j * _NT * _B, _NT * _B)], tileh)
        combine_16x256(tileh, comb)

    def write_dec(j, prefix, r, k):
        stg_i[...] = set_lane(set_lane(set_lane(
            zero_i, 0, prefix), 1, r), 2, k)
        pltpu.sync_copy(stg_i, decs_s.at[pl.ds(j * _L, _L)])

    def read_decs():
        pltpu.sync_copy(decs_s, dec3_b)
        rows = [dec3_b[pl.ds(j * _L, _L)] for j in range(3)]
        return ([lane(rows[j], 0) for j in range(3)],
                [lane(rows[j], 1) for j in range(3)],
                [lane(rows[j], 2) for j in range(3)])

    def lookup_k(nv):
        base = pl.multiple_of(jnp.bitwise_and(nv, _I32(-8)), 8)
        pltpu.sync_copy(ktab_h.at[pl.ds(base, _L)], ktb)
        return lane(ktb[...], nv - base)

    def p1_body(j, _):
        def cbdy(v, _2):
            acc = hist3[pl.ds(j * (_L * _B) + v * _L, _L)]
            for t in range(1, _NT):
                acc = acc + hist3[pl.ds(j * (_L * _B) + t * _B + v * _L, _L)]
            comb[pl.ds(v * _L, _L)] = acc
            return 0
        lax.fori_loop(0, _B // _L, cbdy, 0)
        publish(j)
        return 0
    with jax.named_scope("ph1_hist"):
        lax.fori_loop(0, 3, p1_body, 0)
    plsc.subcore_barrier()

    @pl.when(wid < 3)
    def _():
        j = wid
        pltpu.sync_copy(counts_s, cnt_rb)
        cacc = cnt_rb[pl.ds(0, _L)]
        for t in range(1, _NT):
            cacc = cacc + cnt_rb[pl.ds(t * _L, _L)]
        nv = lane(cacc, j)
        k = lookup_k(nv)
        fetch_and_combine(j)
        b_star, above_star = decide(k)
        write_dec(j, b_star, k - above_star, k)

    zero_hist()
    plsc.subcore_barrier()
    p1s, r1s, k1s = read_decs()

    wcnts = []
    sumhis = []
    _ns2 = jax.named_scope("ph2_hist"); _ns2.__enter__()
    for j in range(3):
        p1j = p1s[j]

        def sb2(i, carry):
            wcnt, shi = carry
            kv = kall[pl.ds(j * _ROWS + i * _L, _L)]
            d1 = jnp.right_shift(kv, 23)
            msk = d1 == p1j
            digit = jnp.bitwise_and(jnp.right_shift(kv, 15), _B - 1)
            plsc.addupdate_scatter(hist, [iota * _B + digit], ones_i, mask=msk)
            plsc.store_compressed(cb.at[pl.ds(j * _CB + wcnt, _L)], kv, mask=msk)
            wcnt = wcnt + plsc.all_reduce_population_count(msk)[0]
            shi = shi + jnp.where(d1 > p1j, plsc.bitcast(kv, _F32), fzero)
            return (wcnt, shi)

        wcnt, shi = lax.fori_loop(0, _ROWS // _L, sb2, (_I32(0), fzero))
        cb[pl.ds(j * _CB + wcnt, _L)] = jnp.full((_L,), -1, _I32)
        wcnts.append(wcnt)
        sumhis.append(shi)
        combine_16x256(hist, comb)
        publish(j)
        if j < 2:
            zero_hist()
    _ns2.__exit__(None, None, None)

    def hist_compacted(j, pj, shift, width):
        trip = jnp.right_shift(wcnts[j] + (_L - 1), 4)
        pshift = shift + width

        def sbc(i, _2):
            kv = cb[pl.ds(j * _CB + i * _L, _L)]
            msk = jnp.right_shift(kv, pshift) == pj
            digit = jnp.bitwise_and(jnp.right_shift(kv, shift), (1 << width) - 1)
            plsc.addupdate_scatter(hist, [iota * _B + digit], ones_i, mask=msk)
            return 0

        lax.fori_loop(0, trip, sbc, 0)
        combine_16x256(hist, comb)
        publish(j)

    plsc.subcore_barrier()

    def decide_round(width):
        @pl.when(wid < 3)
        def _():
            j = wid
            row = dec3_b[pl.ds(wid * _L, _L)]
            p = lane(row, 0)
            r = lane(row, 1)
            k = lane(row, 2)
            fetch_and_combine(j)
            b_star, above_star = decide(r)
            write_dec(j, p * _I32(1 << width) + b_star, r - above_star, k)

    decide_round(8)
    zero_hist()
    plsc.subcore_barrier()
    p2s, r2s, k2s = read_decs()

    for (sh, w) in ((7, 8), (0, 7)):
        for j in range(3):
            hist_compacted(j, p2s[j], sh, w)
            if j < 2:
                zero_hist()
        plsc.subcore_barrier()
        decide_round(w)
        if sh == 7:
            zero_hist()
        plsc.subcore_barrier()
        p2s, r2s, k2s = read_decs()

    ts, rfs, kfs = p2s, r2s, k2s

    svec = fzero
    for j in range(3):
        tj = ts[j]
        trip = jnp.right_shift(wcnts[j] + (_L - 1), 4)

        def sfin(i, acc, j=j, tj=tj):
            kv = cb[pl.ds(j * _CB + i * _L, _L)]
            return acc + jnp.where(kv > tj, plsc.bitcast(kv, _F32), fzero)

        acc = lax.fori_loop(0, trip, sfin, sumhis[j])
        svec = set_lane(svec, j, jnp.sum(acc))
    stg_f[...] = svec
    pltpu.sync_copy(stg_f, sums_s.at[pl.ds(wid * _L, _L)])
    plsc.subcore_barrier()

    @pl.when(wid == 0)
    def _(ts=ts, rfs=rfs, kfs=kfs):
        pltpu.sync_copy(sums_s, sbuf)
        sacc = sbuf[pl.ds(0, _L)]
        for t in range(1, _NT):
            sacc = sacc + sbuf[pl.ds(t * _L, _L)]
        tvec = zero_i
        rvec = zero_i
        kvec = zero_i
        for j in range(3):
            tvec = set_lane(tvec, j, ts[j])
            rvec = set_lane(rvec, j, rfs[j])
            kvec = set_lane(kvec, j, kfs[j])
        total = sacc + rvec.astype(_F32) * plsc.bitcast(tvec, _F32)
        res = total / kvec.astype(_F32)
        res = jnp.where(kvec > 0, res, jnp.full((_L,), jnp.nan, _F32))
        stg_f[...] = res
        pltpu.sync_copy(stg_f, out_h)


def _make_call():
    mesh = plsc.VectorSubcoreMesh(
        core_axis_name="c", subcore_axis_name="s", num_cores=1)
    return pl.kernel(
        _sc_body,
        out_type=jax.ShapeDtypeStruct((_L,), jnp.float32),
        mesh=mesh,
        scratch_types=[
            pltpu.VMEM((_CHUNK,), _I32),
            pltpu.VMEM((2, _CHUNK), _F32),
            pltpu.VMEM((4, _CHUNK), _F32),
            pltpu.VMEM((4, _CHUNK), _F32),
            pltpu.VMEM((10, _CHUNK), _F32),
            pltpu.VMEM((10, _CHUNK), _F32),
            pltpu.VMEM((_CHUNK,), _I32),
            pltpu.VMEM((2, _CHUNK), _F32),
            pltpu.VMEM((4, _CHUNK), _F32),
            pltpu.VMEM((4, _CHUNK), _F32),
            pltpu.VMEM((10, _CHUNK), _F32),
            pltpu.VMEM((10, _CHUNK), _F32),
            pltpu.VMEM((3 * _ROWS,), _I32),
            pltpu.VMEM((3 * _CB,), _I32),
            pltpu.VMEM((_L * _B,), _I32),
            pltpu.VMEM((3 * _L * _B,), _I32),
            pltpu.VMEM((_B,), _I32),
            pltpu.VMEM((_L,), _I32),
            pltpu.VMEM((_L,), _F32),
            pltpu.VMEM((_NT * _B,), _I32),
            pltpu.VMEM((_NT * _L,), _I32),
            pltpu.VMEM((_NT * _L,), _F32),
            pltpu.VMEM((3 * _L,), _I32),
            pltpu.VMEM((_L,), _I32),
            pltpu.VMEM_SHARED((3 * _NT * _B,), _I32),
            pltpu.VMEM_SHARED((_NT * _L,), _I32),
            pltpu.VMEM_SHARED((_NT * _L,), _F32),
            pltpu.VMEM_SHARED((3 * _L,), _I32),
            pltpu.SemaphoreType.DMA,
            pltpu.SemaphoreType.DMA,
        ],
        compiler_params=pltpu.CompilerParams(
            needs_layout_passes=False, use_tc_tiling_on_sc=True),
    )


_sc_call = _make_call()


@jax.jit
def kernel(clss, cls_pred, bbox_true, bbox_pred, ldmk_true, ldmk_pred):
    ktab = jnp.asarray(_KTAB_NP)
    out = _sc_call(
        clss.astype(_I32),
        cls_pred.astype(_F32).T,
        bbox_true.astype(_F32).T,
        bbox_pred.astype(_F32).T,
        ldmk_true.astype(_F32).T,
        ldmk_pred.astype(_F32).T,
        ktab,
    )
    return out[:3]
